# baseline (speedup 1.0000x reference)
import functools

import jax
import jax.numpy as jnp
from jax import lax
from jax.experimental import pallas as pl
from jax.experimental.pallas import tpu as pltpu
from jax.experimental.pallas import tpu_sc as plsc

NUM_BINS = 256
RANGE_MIN = -5.0
RANGE_MAX = 5.0
STEP = (RANGE_MAX - RANGE_MIN) / NUM_BINS

SHAPE = (4, 4096, 2048)
SLAB_ROWS = SHAPE[1]
COLS = SHAPE[2]
NUM_CORES = 2
NUM_SUBCORES = 16
NW = NUM_CORES * NUM_SUBCORES
ROWS_PER_CHUNK = 8
ROWS_PER_W = SLAB_ROWS // NW
NCHUNK = ROWS_PER_W // ROWS_PER_CHUNK
NCB = COLS // 16
NBUF = 2

_noise_cache = []


def _noise_slabs():
    if not _noise_cache:
        with jax.ensure_compile_time_eval():
            n = jax.random.normal(jax.random.key(1), SHAPE,
                                  dtype=jnp.float32) * 0.01
            slabs = [jax.block_until_ready(n[a]) for a in range(SHAPE[0])]
        _noise_cache.append(slabs)
    return _noise_cache[0]


@functools.lru_cache(maxsize=None)
def _sc_gather_fn(slab):
    mesh = plsc.VectorSubcoreMesh(
        core_axis_name="c", subcore_axis_name="s",
        num_cores=NUM_CORES, num_subcores=NUM_SUBCORES,
    )

    @functools.partial(
        pl.kernel,
        out_type=jax.ShapeDtypeStruct((SLAB_ROWS, COLS), jnp.float32),
        mesh=mesh,
        compiler_params=pltpu.CompilerParams(needs_layout_passes=False),
        scratch_types=[
            pltpu.VMEM((2, 128), jnp.float32),
            pltpu.VMEM((ROWS_PER_CHUNK, COLS), jnp.float32),
            pltpu.VMEM((ROWS_PER_CHUNK, COLS), jnp.float32),
            pltpu.VMEM((ROWS_PER_CHUNK, COLS), jnp.float32),
            pltpu.VMEM((ROWS_PER_CHUNK, COLS), jnp.float32),
            pltpu.SemaphoreType.DMA((NBUF,)),
            pltpu.SemaphoreType.DMA((NBUF,)),
        ],
    )
    def fn(x_hbm, bins_hbm, out_hbm,
           binsv, xb0, xb1, ob0, ob1, semx, semo):
        wid = lax.axis_index("s") * NUM_CORES + lax.axis_index("c")
        row_base = wid * ROWS_PER_W
        xbs = (xb0, xb1)
        obs = (ob0, ob1)

        pltpu.sync_copy(bins_hbm, binsv)

        def row0(chunk):
            return pl.multiple_of(row_base + chunk * ROWS_PER_CHUNK,
                                  ROWS_PER_CHUNK)

        def start_in(chunk, b):
            r = row0(chunk)
            pltpu.async_copy(x_hbm.at[slab, pl.ds(r, ROWS_PER_CHUNK), :],
                             xbs[b], semx.at[b])

        def wait_in(chunk, b):
            r = row0(chunk)
            pltpu.make_async_copy(x_hbm.at[slab, pl.ds(r, ROWS_PER_CHUNK), :],
                                  xbs[b], semx.at[b]).wait()

        def start_out(chunk, b):
            r = row0(chunk)
            pltpu.async_copy(obs[b], out_hbm.at[pl.ds(r, ROWS_PER_CHUNK), :],
                             semo.at[b])

        def wait_out(chunk, b):
            r = row0(chunk)
            pltpu.make_async_copy(obs[b],
                                  out_hbm.at[pl.ds(r, ROWS_PER_CHUNK), :],
                                  semo.at[b]).wait()

        def compute(b):
            xb, ob = xbs[b], obs[b]

            @plsc.parallel_loop(0, NCB, unroll=2)
            def vbody(v):
                off = pl.multiple_of(v * 16, 16)
                for r in range(ROWS_PER_CHUNK):
                    xv = xb[r, pl.ds(off, 16)]
                    t = (xv - RANGE_MIN) / STEP
                    idx = jnp.clip(t.astype(jnp.int32), 0, NUM_BINS - 1)
                    g = plsc.load_gather(
                        binsv,
                        [lax.shift_right_logical(idx, 7), idx & 127])
                    ob[r, pl.ds(off, 16)] = g

        for b in range(NBUF):
            start_in(b, b)

        def outer(g2, carry):
            for b in range(NBUF):
                chunk = g2 * NBUF + b
                wait_in(chunk, b)

                @pl.when(g2 >= 1)
                def _():
                    wait_out(chunk - NBUF, b)

                compute(b)
                start_out(chunk, b)

                @pl.when(g2 < NCHUNK // NBUF - 1)
                def _():
                    start_in(chunk + NBUF, b)
            return carry

        lax.fori_loop(0, NCHUNK // NBUF, outer, 0)

        for b in range(NBUF):
            wait_out(NCHUNK - NBUF + b, b)

    return fn


def kernel(x, bins):
    noise = _noise_slabs()
    bins2 = bins.reshape(2, 128)
    gathered = [_sc_gather_fn(a)(x, bins2) for a in range(SHAPE[0])]
    return jnp.stack([g + noise[a] for a, g in enumerate(gathered)])


# speedup vs baseline: 1.4090x; 1.4090x over previous
import functools

import jax
import jax.numpy as jnp
from jax import lax
from jax.experimental import pallas as pl
from jax.experimental.pallas import tpu as pltpu
from jax.experimental.pallas import tpu_sc as plsc

NUM_BINS = 256
RANGE_MIN = -5.0
RANGE_MAX = 5.0
STEP = (RANGE_MAX - RANGE_MIN) / NUM_BINS

SHAPE = (4, 4096, 2048)
TOTAL = SHAPE[0] * SHAPE[1] * SHAPE[2]
NUM_CORES = 2
NUM_SUBCORES = 16
NW = NUM_CORES * NUM_SUBCORES
ROWS_PER_CHUNK = 8
COLS = SHAPE[2]
NCHUNK = SHAPE[1] // ROWS_PER_CHUNK // (NW // SHAPE[0])
NCB = COLS // 16
NBUF = 2

_noise_cache = []


def _noise3d():
    if not _noise_cache:
        with jax.ensure_compile_time_eval():
            n = jax.random.normal(jax.random.key(1), SHAPE,
                                  dtype=jnp.float32) * 0.01
        _noise_cache.append(jax.block_until_ready(n))
    return _noise_cache[0]


@functools.lru_cache(maxsize=None)
def _sc_fn():
    mesh = plsc.VectorSubcoreMesh(
        core_axis_name="c", subcore_axis_name="s",
        num_cores=NUM_CORES, num_subcores=NUM_SUBCORES,
    )

    @functools.partial(
        pl.kernel,
        out_type=jax.ShapeDtypeStruct(SHAPE, jnp.float32),
        mesh=mesh,
        compiler_params=pltpu.CompilerParams(needs_layout_passes=False),
        scratch_types=[
            pltpu.VMEM((2, 128), jnp.float32),
            pltpu.VMEM((ROWS_PER_CHUNK, COLS), jnp.float32),
            pltpu.VMEM((ROWS_PER_CHUNK, COLS), jnp.float32),
            pltpu.VMEM((ROWS_PER_CHUNK, COLS), jnp.float32),
            pltpu.VMEM((ROWS_PER_CHUNK, COLS), jnp.float32),
            pltpu.VMEM((ROWS_PER_CHUNK, COLS), jnp.float32),
            pltpu.VMEM((ROWS_PER_CHUNK, COLS), jnp.float32),
            pltpu.SemaphoreType.DMA((NBUF,)),
            pltpu.SemaphoreType.DMA((NBUF,)),
            pltpu.SemaphoreType.DMA((NBUF,)),
        ],
    )
    def fn(x_hbm, noise_hbm, bins_hbm, out_hbm,
           binsv, xb0, xb1, nb0, nb1, ob0, ob1, semx, semn, semo):
        wid = lax.axis_index("s") * NUM_CORES + lax.axis_index("c")
        a = lax.shift_right_logical(wid, 3)
        row_base = lax.shift_left(wid & 7, 9)
        xbs = (xb0, xb1)
        nbs = (nb0, nb1)
        obs = (ob0, ob1)

        pltpu.sync_copy(bins_hbm, binsv)

        def row0(chunk):
            return pl.multiple_of(row_base + chunk * ROWS_PER_CHUNK,
                                  ROWS_PER_CHUNK)

        def start_in(chunk, b):
            r = row0(chunk)
            pltpu.async_copy(x_hbm.at[a, pl.ds(r, ROWS_PER_CHUNK), :],
                             xbs[b], semx.at[b])
            pltpu.async_copy(noise_hbm.at[a, pl.ds(r, ROWS_PER_CHUNK), :],
                             nbs[b], semn.at[b])

        def wait_in(chunk, b):
            r = row0(chunk)
            pltpu.make_async_copy(x_hbm.at[a, pl.ds(r, ROWS_PER_CHUNK), :],
                                  xbs[b], semx.at[b]).wait()
            pltpu.make_async_copy(noise_hbm.at[a, pl.ds(r, ROWS_PER_CHUNK), :],
                                  nbs[b], semn.at[b]).wait()

        def start_out(chunk, b):
            r = row0(chunk)
            pltpu.async_copy(obs[b], out_hbm.at[a, pl.ds(r, ROWS_PER_CHUNK), :],
                             semo.at[b])

        def wait_out(chunk, b):
            r = row0(chunk)
            pltpu.make_async_copy(obs[b],
                                  out_hbm.at[a, pl.ds(r, ROWS_PER_CHUNK), :],
                                  semo.at[b]).wait()

        def compute(b):
            xb, nb, ob = xbs[b], nbs[b], obs[b]

            @plsc.parallel_loop(0, NCB, unroll=2)
            def vbody(v):
                off = pl.multiple_of(v * 16, 16)
                for r in range(ROWS_PER_CHUNK):
                    xv = xb[r, pl.ds(off, 16)]
                    t = (xv - RANGE_MIN) / STEP
                    idx = jnp.clip(t.astype(jnp.int32), 0, NUM_BINS - 1)
                    g = plsc.load_gather(
                        binsv,
                        [lax.shift_right_logical(idx, 7), idx & 127])
                    ob[r, pl.ds(off, 16)] = g + nb[r, pl.ds(off, 16)]

        for b in range(NBUF):
            start_in(b, b)

        def outer(g2, carry):
            for b in range(NBUF):
                chunk = g2 * NBUF + b
                wait_in(chunk, b)

                @pl.when(g2 >= 1)
                def _():
                    wait_out(chunk - NBUF, b)

                compute(b)
                start_out(chunk, b)

                @pl.when(g2 < NCHUNK // NBUF - 1)
                def _():
                    start_in(chunk + NBUF, b)
            return carry

        lax.fori_loop(0, NCHUNK // NBUF, outer, 0)

        for b in range(NBUF):
            wait_out(NCHUNK - NBUF + b, b)

    return fn


def kernel(x, bins):
    return _sc_fn()(x, _noise3d(), bins.reshape(2, 128))
